# baseline (speedup 1.0000x reference)
import functools

import jax
import jax.numpy as jnp
from jax import lax
from jax.experimental import pallas as pl
from jax.experimental.pallas import tpu as pltpu
from jax.experimental.pallas import tpu_sc as plsc

NC = 2
NS = 16
NW = NC * NS
L = 16
NSUB = 4


@functools.lru_cache(maxsize=None)
def _face_normals_sc(N, V):
    CH = -(-N // (NW * 128)) * 128
    CT = N - (NW - 1) * CH
    CHS = CH // NSUB
    tail_sz = [min(CHS, max(CT - j * CHS, 0)) for j in range(NSUB)]
    assert CH % 128 == 0 and 0 < CT <= CH and CT % 8 == 0
    assert all(t % 8 == 0 for t in tail_sz)
    VP = -(-V // (NS * 8)) * (NS * 8)
    SEG = VP // NS
    VT = V - SEG * (NS - 1)
    assert 0 < VT <= SEG and VT % 8 == 0
    mesh = plsc.VectorSubcoreMesh(core_axis_name="c", subcore_axis_name="s")
    out_t = [jax.ShapeDtypeStruct((N,), jnp.float32)] * 3
    scratch = (
        [pltpu.VMEM_SHARED((VP,), jnp.float32)] * 3
        + [pltpu.VMEM((CH,), jnp.int32)] * 3
        + [pltpu.VMEM((CHS,), jnp.float32)] * 18
        + [pltpu.VMEM((CH,), jnp.float32)] * 3
        + [pltpu.SemaphoreType.DMA] * 2
    )

    @functools.partial(
        pl.kernel, mesh=mesh, out_type=out_t, scratch_types=scratch,
        compiler_params=pltpu.CompilerParams(needs_layout_passes=False))
    def k(vx, vy, vz, f0, f1, f2, onx, ony, onz, *refs):
        (svx, svy, svz, i0, i1, i2) = refs[:6]
        bufs = (refs[6:15], refs[15:24])
        (ox, oy, oz) = refs[24:27]
        sems = refs[27:29]
        sid = lax.axis_index("s")
        wid = sid * NC + lax.axis_index("c")
        base = wid * CH
        tail = wid == NW - 1

        voff = sid * SEG

        def stage(n):
            b0, b1, b2 = bufs[0][0], bufs[0][3], bufs[0][6]
            for src, dst, b in ((vx, svx, b0), (vy, svy, b1), (vz, svz, b2)):
                done = 0
                while done < n:
                    sz = min(CHS, n - done)
                    pltpu.sync_copy(src.at[pl.ds(voff + done, sz)],
                                    b.at[pl.ds(0, sz)])
                    pltpu.sync_copy(b.at[pl.ds(0, sz)],
                                    dst.at[pl.ds(voff + done, sz)])
                    done += sz

        @pl.when(sid < NS - 1)
        def _stage_full():
            stage(SEG)

        @pl.when(sid == NS - 1)
        def _stage_tail():
            stage(VT)

        @pl.when(jnp.logical_not(tail))
        def _idx_full():
            pltpu.sync_copy(f0.at[pl.ds(base, CH)], i0)
            pltpu.sync_copy(f1.at[pl.ds(base, CH)], i1)
            pltpu.sync_copy(f2.at[pl.ds(base, CH)], i2)

        @pl.when(tail)
        def _idx_tail():
            pltpu.sync_copy(f0.at[pl.ds(base, CT)], i0.at[pl.ds(0, CT)])
            pltpu.sync_copy(f1.at[pl.ds(base, CT)], i1.at[pl.ds(0, CT)])
            pltpu.sync_copy(f2.at[pl.ds(base, CT)], i2.at[pl.ds(0, CT)])

        plsc.subcore_barrier()

        def fire(j):
            b = bufs[j % 2]
            sem = sems[j % 2]
            off = j * CHS
            fsz = tail_sz[j]

            def launch(sz):
                for t, (idx, tab) in enumerate(
                        ((i0, svx), (i0, svy), (i0, svz),
                         (i1, svx), (i1, svy), (i1, svz),
                         (i2, svx), (i2, svy), (i2, svz))):
                    pltpu.async_copy(tab.at[idx.at[pl.ds(off, sz)]],
                                     b[t].at[pl.ds(0, sz)], sem)

            if fsz == CHS:
                launch(CHS)
            else:
                @pl.when(jnp.logical_not(tail))
                def _f():
                    launch(CHS)

                if fsz > 0:
                    @pl.when(tail)
                    def _t():
                        launch(fsz)

        def drain(j):
            b = bufs[j % 2]
            sem = sems[j % 2]
            fsz = tail_sz[j]

            def dr(sz):
                for t in range(9):
                    pltpu.make_async_copy(vx.at[pl.ds(0, sz)],
                                          b[t].at[pl.ds(0, sz)], sem).wait()

            if fsz == CHS:
                dr(CHS)
            else:
                @pl.when(jnp.logical_not(tail))
                def _f():
                    dr(CHS)

                if fsz > 0:
                    @pl.when(tail)
                    def _t():
                        dr(fsz)

        def compute(j):
            (x0, y0, z0, x1, y1, z1, x2, y2, z2) = bufs[j % 2]
            obase = j * CHS

            def step(i, carry):
                s = pl.ds(i * L, L)
                so = pl.ds(obase + i * L, L)
                ax0 = x0[s]; ay0 = y0[s]; az0 = z0[s]
                ax1 = x1[s]; ay1 = y1[s]; az1 = z1[s]
                ax2 = x2[s]; ay2 = y2[s]; az2 = z2[s]
                e1x = ax0 - ax1; e1y = ay0 - ay1; e1z = az0 - az1
                e2x = ax2 - ax1; e2y = ay2 - ay1; e2z = az2 - az1
                nx = e2y * e1z - e2z * e1y
                ny = e2z * e1x - e2x * e1z
                nz = e2x * e1y - e2y * e1x
                nn = nx * nx + ny * ny + nz * nz
                ii = jnp.int32(0x5F3759DF) - (plsc.bitcast(nn, jnp.int32) >> 1)
                r = plsc.bitcast(ii, jnp.float32)
                h = nn * jnp.float32(0.5)
                r = r * (jnp.float32(1.5) - (h * r) * r)
                r = r * (jnp.float32(1.5) - (h * r) * r)
                ox[so] = nx * r
                oy[so] = ny * r
                oz[so] = nz * r
                return carry

            lax.fori_loop(0, CHS // L, step, 0, unroll=7)

        fire(0)
        for j in range(NSUB):
            if j + 1 < NSUB:
                fire(j + 1)
            drain(j)
            compute(j)

        @pl.when(jnp.logical_not(tail))
        def _out_full():
            pltpu.sync_copy(ox, onx.at[pl.ds(base, CH)])
            pltpu.sync_copy(oy, ony.at[pl.ds(base, CH)])
            pltpu.sync_copy(oz, onz.at[pl.ds(base, CH)])

        @pl.when(tail)
        def _out_tail():
            pltpu.sync_copy(ox.at[pl.ds(0, CT)], onx.at[pl.ds(base, CT)])
            pltpu.sync_copy(oy.at[pl.ds(0, CT)], ony.at[pl.ds(base, CT)])
            pltpu.sync_copy(oz.at[pl.ds(0, CT)], onz.at[pl.ds(base, CT)])

    return k


def kernel(vertices, faces):
    fi = faces.astype(jnp.int32)
    N = fi.shape[0]
    V = vertices.shape[0]
    onx, ony, onz = _face_normals_sc(N, V)(
        vertices[:, 0], vertices[:, 1], vertices[:, 2],
        fi[:, 0], fi[:, 1], fi[:, 2])
    return jnp.stack([onx, ony, onz], axis=-1)


# speedup vs baseline: 1.0021x; 1.0021x over previous
import functools

import jax
import jax.numpy as jnp
from jax import lax
from jax.experimental import pallas as pl
from jax.experimental.pallas import tpu as pltpu
from jax.experimental.pallas import tpu_sc as plsc

NC = 2
NS = 16
NW = NC * NS
L = 16
NSUB = 4


@functools.lru_cache(maxsize=None)
def _face_normals_sc(N, V):
    CH = -(-N // (NW * 128)) * 128
    CT = N - (NW - 1) * CH
    CHS = CH // NSUB
    tail_sz = [min(CHS, max(CT - j * CHS, 0)) for j in range(NSUB)]
    assert CH % 128 == 0 and 0 < CT <= CH and CT % 8 == 0
    assert all(t % 8 == 0 for t in tail_sz)
    VP = -(-V // (NS * 8)) * (NS * 8)
    SEG = VP // NS
    VT = V - SEG * (NS - 1)
    assert 0 < VT <= SEG and VT % 8 == 0
    mesh = plsc.VectorSubcoreMesh(core_axis_name="c", subcore_axis_name="s")
    out_t = [jax.ShapeDtypeStruct((N,), jnp.float32)] * 3
    scratch = (
        [pltpu.VMEM_SHARED((VP,), jnp.float32)] * 3
        + [pltpu.VMEM((CH,), jnp.int32)] * 3
        + [pltpu.VMEM((CHS,), jnp.float32)] * 18
        + [pltpu.VMEM((CH,), jnp.float32)] * 3
        + [pltpu.SemaphoreType.DMA] * 2
    )

    @functools.partial(
        pl.kernel, mesh=mesh, out_type=out_t, scratch_types=scratch,
        compiler_params=pltpu.CompilerParams(needs_layout_passes=False,
                                             skip_device_barrier=True,
                                             disable_bounds_checks=True,
                                             disable_semaphore_checks=True))
    def k(vx, vy, vz, f0, f1, f2, onx, ony, onz, *refs):
        (svx, svy, svz, i0, i1, i2) = refs[:6]
        bufs = (refs[6:15], refs[15:24])
        (ox, oy, oz) = refs[24:27]
        sems = refs[27:29]
        sid = lax.axis_index("s")
        wid = sid * NC + lax.axis_index("c")
        base = wid * CH
        tail = wid == NW - 1

        voff = sid * SEG

        def stage(n):
            b0, b1, b2 = bufs[0][0], bufs[0][3], bufs[0][6]
            for src, dst, b in ((vx, svx, b0), (vy, svy, b1), (vz, svz, b2)):
                done = 0
                while done < n:
                    sz = min(CHS, n - done)
                    pltpu.sync_copy(src.at[pl.ds(voff + done, sz)],
                                    b.at[pl.ds(0, sz)])
                    pltpu.sync_copy(b.at[pl.ds(0, sz)],
                                    dst.at[pl.ds(voff + done, sz)])
                    done += sz

        @pl.when(sid < NS - 1)
        def _stage_full():
            stage(SEG)

        @pl.when(sid == NS - 1)
        def _stage_tail():
            stage(VT)

        @pl.when(jnp.logical_not(tail))
        def _idx_full():
            pltpu.sync_copy(f0.at[pl.ds(base, CH)], i0)
            pltpu.sync_copy(f1.at[pl.ds(base, CH)], i1)
            pltpu.sync_copy(f2.at[pl.ds(base, CH)], i2)

        @pl.when(tail)
        def _idx_tail():
            pltpu.sync_copy(f0.at[pl.ds(base, CT)], i0.at[pl.ds(0, CT)])
            pltpu.sync_copy(f1.at[pl.ds(base, CT)], i1.at[pl.ds(0, CT)])
            pltpu.sync_copy(f2.at[pl.ds(base, CT)], i2.at[pl.ds(0, CT)])

        plsc.subcore_barrier()

        def fire(j):
            b = bufs[j % 2]
            sem = sems[j % 2]
            off = j * CHS
            fsz = tail_sz[j]

            def launch(sz):
                for t, (idx, tab) in enumerate(
                        ((i0, svx), (i0, svy), (i0, svz),
                         (i1, svx), (i1, svy), (i1, svz),
                         (i2, svx), (i2, svy), (i2, svz))):
                    pltpu.async_copy(tab.at[idx.at[pl.ds(off, sz)]],
                                     b[t].at[pl.ds(0, sz)], sem)

            if fsz == CHS:
                launch(CHS)
            else:
                @pl.when(jnp.logical_not(tail))
                def _f():
                    launch(CHS)

                if fsz > 0:
                    @pl.when(tail)
                    def _t():
                        launch(fsz)

        def drain(j):
            b = bufs[j % 2]
            sem = sems[j % 2]
            fsz = tail_sz[j]

            def dr(sz):
                for t in range(9):
                    pltpu.make_async_copy(vx.at[pl.ds(0, sz)],
                                          b[t].at[pl.ds(0, sz)], sem).wait()

            if fsz == CHS:
                dr(CHS)
            else:
                @pl.when(jnp.logical_not(tail))
                def _f():
                    dr(CHS)

                if fsz > 0:
                    @pl.when(tail)
                    def _t():
                        dr(fsz)

        def compute(j):
            (x0, y0, z0, x1, y1, z1, x2, y2, z2) = bufs[j % 2]
            obase = j * CHS

            def step(i, carry):
                s = pl.ds(i * L, L)
                so = pl.ds(obase + i * L, L)
                ax0 = x0[s]; ay0 = y0[s]; az0 = z0[s]
                ax1 = x1[s]; ay1 = y1[s]; az1 = z1[s]
                ax2 = x2[s]; ay2 = y2[s]; az2 = z2[s]
                e1x = ax0 - ax1; e1y = ay0 - ay1; e1z = az0 - az1
                e2x = ax2 - ax1; e2y = ay2 - ay1; e2z = az2 - az1
                nx = e2y * e1z - e2z * e1y
                ny = e2z * e1x - e2x * e1z
                nz = e2x * e1y - e2y * e1x
                nn = nx * nx + ny * ny + nz * nz
                ii = jnp.int32(0x5F3759DF) - (plsc.bitcast(nn, jnp.int32) >> 1)
                r = plsc.bitcast(ii, jnp.float32)
                h = nn * jnp.float32(0.5)
                r = r * (jnp.float32(1.5) - (h * r) * r)
                r = r * (jnp.float32(1.5) - (h * r) * r)
                ox[so] = nx * r
                oy[so] = ny * r
                oz[so] = nz * r
                return carry

            lax.fori_loop(0, CHS // L, step, 0, unroll=7)

        fire(0)
        for j in range(NSUB):
            if j + 1 < NSUB:
                fire(j + 1)
            drain(j)
            compute(j)

        @pl.when(jnp.logical_not(tail))
        def _out_full():
            pltpu.sync_copy(ox, onx.at[pl.ds(base, CH)])
            pltpu.sync_copy(oy, ony.at[pl.ds(base, CH)])
            pltpu.sync_copy(oz, onz.at[pl.ds(base, CH)])

        @pl.when(tail)
        def _out_tail():
            pltpu.sync_copy(ox.at[pl.ds(0, CT)], onx.at[pl.ds(base, CT)])
            pltpu.sync_copy(oy.at[pl.ds(0, CT)], ony.at[pl.ds(base, CT)])
            pltpu.sync_copy(oz.at[pl.ds(0, CT)], onz.at[pl.ds(base, CT)])

    return k


def kernel(vertices, faces):
    fi = faces.astype(jnp.int32)
    N = fi.shape[0]
    V = vertices.shape[0]
    onx, ony, onz = _face_normals_sc(N, V)(
        vertices[:, 0], vertices[:, 1], vertices[:, 2],
        fi[:, 0], fi[:, 1], fi[:, 2])
    return jnp.stack([onx, ony, onz], axis=-1)


# speedup vs baseline: 1.0087x; 1.0066x over previous
import functools

import jax
import jax.numpy as jnp
from jax import lax
from jax.experimental import pallas as pl
from jax.experimental.pallas import tpu as pltpu
from jax.experimental.pallas import tpu_sc as plsc

NC = 2
NS = 16
NW = NC * NS
L = 16
NSUB = 4


@functools.lru_cache(maxsize=None)
def _face_normals_sc(N, V):
    CH = -(-N // (NW * 128)) * 128
    CT = N - (NW - 1) * CH
    CHS = CH // NSUB
    tail_sz = [min(CHS, max(CT - j * CHS, 0)) for j in range(NSUB)]
    assert CH % 128 == 0 and 0 < CT <= CH and CT % 8 == 0
    assert all(t % 8 == 0 for t in tail_sz)
    VP = -(-V // (NS * 8)) * (NS * 8)
    SEG = VP // NS
    VT = V - SEG * (NS - 1)
    assert 0 < VT <= SEG and VT % 8 == 0
    mesh = plsc.VectorSubcoreMesh(core_axis_name="c", subcore_axis_name="s")
    out_t = [jax.ShapeDtypeStruct((N,), jnp.float32)] * 3
    scratch = (
        [pltpu.VMEM_SHARED((VP,), jnp.float32)] * 3
        + [pltpu.VMEM((CH,), jnp.int32)] * 3
        + [pltpu.VMEM((CHS,), jnp.float32)] * 18
        + [pltpu.VMEM((CH,), jnp.float32)] * 3
        + [pltpu.SemaphoreType.DMA] * 2
    )

    @functools.partial(
        pl.kernel, mesh=mesh, out_type=out_t, scratch_types=scratch,
        compiler_params=pltpu.CompilerParams(needs_layout_passes=False,
                                             use_tc_tiling_on_sc=False,
                                             skip_device_barrier=True,
                                             disable_bounds_checks=True,
                                             disable_semaphore_checks=True))
    def k(vx, vy, vz, f0, f1, f2, onx, ony, onz, *refs):
        (svx, svy, svz, i0, i1, i2) = refs[:6]
        bufs = (refs[6:15], refs[15:24])
        (ox, oy, oz) = refs[24:27]
        sems = refs[27:29]
        sid = lax.axis_index("s")
        wid = sid * NC + lax.axis_index("c")
        base = wid * CH
        tail = wid == NW - 1

        voff = sid * SEG

        def stage(n):
            b0, b1, b2 = bufs[0][0], bufs[0][3], bufs[0][6]
            for src, dst, b in ((vx, svx, b0), (vy, svy, b1), (vz, svz, b2)):
                done = 0
                while done < n:
                    sz = min(CHS, n - done)
                    pltpu.sync_copy(src.at[pl.ds(voff + done, sz)],
                                    b.at[pl.ds(0, sz)])
                    pltpu.sync_copy(b.at[pl.ds(0, sz)],
                                    dst.at[pl.ds(voff + done, sz)])
                    done += sz

        @pl.when(sid < NS - 1)
        def _stage_full():
            stage(SEG)

        @pl.when(sid == NS - 1)
        def _stage_tail():
            stage(VT)

        @pl.when(jnp.logical_not(tail))
        def _idx_full():
            pltpu.sync_copy(f0.at[pl.ds(base, CH)], i0)
            pltpu.sync_copy(f1.at[pl.ds(base, CH)], i1)
            pltpu.sync_copy(f2.at[pl.ds(base, CH)], i2)

        @pl.when(tail)
        def _idx_tail():
            pltpu.sync_copy(f0.at[pl.ds(base, CT)], i0.at[pl.ds(0, CT)])
            pltpu.sync_copy(f1.at[pl.ds(base, CT)], i1.at[pl.ds(0, CT)])
            pltpu.sync_copy(f2.at[pl.ds(base, CT)], i2.at[pl.ds(0, CT)])

        plsc.subcore_barrier()

        def fire(j):
            b = bufs[j % 2]
            sem = sems[j % 2]
            off = j * CHS
            fsz = tail_sz[j]

            def launch(sz):
                for t, (idx, tab) in enumerate(
                        ((i0, svx), (i0, svy), (i0, svz),
                         (i1, svx), (i1, svy), (i1, svz),
                         (i2, svx), (i2, svy), (i2, svz))):
                    pltpu.async_copy(tab.at[idx.at[pl.ds(off, sz)]],
                                     b[t].at[pl.ds(0, sz)], sem)

            if fsz == CHS:
                launch(CHS)
            else:
                @pl.when(jnp.logical_not(tail))
                def _f():
                    launch(CHS)

                if fsz > 0:
                    @pl.when(tail)
                    def _t():
                        launch(fsz)

        def drain(j):
            b = bufs[j % 2]
            sem = sems[j % 2]
            fsz = tail_sz[j]

            def dr(sz):
                for t in range(9):
                    pltpu.make_async_copy(vx.at[pl.ds(0, sz)],
                                          b[t].at[pl.ds(0, sz)], sem).wait()

            if fsz == CHS:
                dr(CHS)
            else:
                @pl.when(jnp.logical_not(tail))
                def _f():
                    dr(CHS)

                if fsz > 0:
                    @pl.when(tail)
                    def _t():
                        dr(fsz)

        def compute(j):
            (x0, y0, z0, x1, y1, z1, x2, y2, z2) = bufs[j % 2]
            obase = j * CHS

            def step(i, carry):
                s = pl.ds(i * L, L)
                so = pl.ds(obase + i * L, L)
                ax0 = x0[s]; ay0 = y0[s]; az0 = z0[s]
                ax1 = x1[s]; ay1 = y1[s]; az1 = z1[s]
                ax2 = x2[s]; ay2 = y2[s]; az2 = z2[s]
                e1x = ax0 - ax1; e1y = ay0 - ay1; e1z = az0 - az1
                e2x = ax2 - ax1; e2y = ay2 - ay1; e2z = az2 - az1
                nx = e2y * e1z - e2z * e1y
                ny = e2z * e1x - e2x * e1z
                nz = e2x * e1y - e2y * e1x
                nn = nx * nx + ny * ny + nz * nz
                ii = jnp.int32(0x5F3759DF) - (plsc.bitcast(nn, jnp.int32) >> 1)
                r = plsc.bitcast(ii, jnp.float32)
                h = nn * jnp.float32(0.5)
                r = r * (jnp.float32(1.5) - (h * r) * r)
                r = r * (jnp.float32(1.5) - (h * r) * r)
                ox[so] = nx * r
                oy[so] = ny * r
                oz[so] = nz * r
                return carry

            lax.fori_loop(0, CHS // L, step, 0, unroll=7)

        fire(0)
        for j in range(NSUB):
            if j + 1 < NSUB:
                fire(j + 1)
            drain(j)
            compute(j)

        @pl.when(jnp.logical_not(tail))
        def _out_full():
            pltpu.sync_copy(ox, onx.at[pl.ds(base, CH)])
            pltpu.sync_copy(oy, ony.at[pl.ds(base, CH)])
            pltpu.sync_copy(oz, onz.at[pl.ds(base, CH)])

        @pl.when(tail)
        def _out_tail():
            pltpu.sync_copy(ox.at[pl.ds(0, CT)], onx.at[pl.ds(base, CT)])
            pltpu.sync_copy(oy.at[pl.ds(0, CT)], ony.at[pl.ds(base, CT)])
            pltpu.sync_copy(oz.at[pl.ds(0, CT)], onz.at[pl.ds(base, CT)])

    return k


def kernel(vertices, faces):
    fi = faces.astype(jnp.int32)
    N = fi.shape[0]
    V = vertices.shape[0]
    onx, ony, onz = _face_normals_sc(N, V)(
        vertices[:, 0], vertices[:, 1], vertices[:, 2],
        fi[:, 0], fi[:, 1], fi[:, 2])
    return jnp.stack([onx, ony, onz], axis=-1)
